# baseline (speedup 1.0000x reference)
import functools

import jax
import jax.numpy as jnp
from jax import lax
from jax.experimental import pallas as pl
from jax.experimental.pallas import tpu as pltpu
from jax.experimental.pallas import tpu_sc as plsc

_N = 10000
_NP = 10240
_E = 320000
_EH = 160000
_EP = 327680
_NW = 32
_NCH = 80
_K = 128
_EPW = _NCH * _K
_HID = 128
_C = 16
_RPT = _NP // 16

_mesh = plsc.VectorSubcoreMesh(core_axis_name="c", subcore_axis_name="s")


def _edge_mlp(Q, W1, b1, W2, b2):
    QT = Q.T
    W1T = W1.T
    W2T = W2.T
    G = 5
    R = _EH // G

    def body(q_ref, w1_ref, b1_ref, w2_ref, b2_ref, o_ref):
        h = jnp.dot(w1_ref[...], q_ref[...], preferred_element_type=jnp.float32)
        h = jnp.maximum(h + b1_ref[...][:, None], 0.0)
        o = jnp.dot(w2_ref[...], h, preferred_element_type=jnp.float32)
        o_ref[...] = jax.nn.sigmoid(o + b2_ref[0])

    out = pl.pallas_call(
        body,
        grid=(G,),
        in_specs=[
            pl.BlockSpec((_C, R), lambda i: (0, i)),
            pl.BlockSpec((32, _C), lambda i: (0, 0)),
            pl.BlockSpec((32,), lambda i: (0,)),
            pl.BlockSpec((1, 32), lambda i: (0, 0)),
            pl.BlockSpec((1,), lambda i: (0,)),
        ],
        out_specs=pl.BlockSpec((1, R), lambda i: (0, i)),
        out_shape=jax.ShapeDtypeStruct((1, _EH), jnp.float32),
    )(QT, W1T, b1, W2T, b2)
    return out[0]


def _dense_matmul(xp, w):
    H = w.shape[1]

    def body(x_ref, w_ref, o_ref):
        o_ref[...] = jnp.dot(x_ref[...], w_ref[...],
                             preferred_element_type=jnp.float32)

    return pl.pallas_call(
        body,
        out_shape=jax.ShapeDtypeStruct((_NP, H), jnp.float32),
    )(xp, w)


def _dinv_and_scale(degp, xw):
    def body(d_ref, xw_ref, dinv_ref, h_ref):
        deg = d_ref[0] + d_ref[1]
        dinv = jnp.where(deg > 0,
                         lax.rsqrt(jnp.maximum(deg, 1e-12)),
                         0.0)
        dinv_ref[...] = dinv
        h_ref[...] = xw_ref[...] * dinv[:, None]

    return pl.pallas_call(
        body,
        out_shape=[
            jax.ShapeDtypeStruct((_NP,), jnp.float32),
            jax.ShapeDtypeStruct((_NP, _HID), jnp.float32),
        ],
    )(degp, xw)


def _layer2_dense(acc1, dinv, c1b, w2):
    def body(a_ref, d_ref, b_ref, w_ref, o_ref):
        s = a_ref[0] + a_ref[1]
        dinv = d_ref[...]
        h = jnp.maximum(s * dinv[:, None] + b_ref[...][None, :], 0.0)
        h2 = jnp.dot(h, w_ref[...], preferred_element_type=jnp.float32)
        o_ref[...] = h2 * dinv[:, None]

    return pl.pallas_call(
        body,
        out_shape=jax.ShapeDtypeStruct((_NP, _C), jnp.float32),
    )(acc1, dinv, c1b, w2)


def _final_out(acc2, dinv, c2b):
    def body(a_ref, d_ref, b_ref, o_ref):
        s = a_ref[0] + a_ref[1]
        o_ref[...] = s * d_ref[...][:, None] + b_ref[...][None, :]

    return pl.pallas_call(
        body,
        out_shape=jax.ShapeDtypeStruct((_NP, _C), jnp.float32),
    )(acc2, dinv, c2b)


@functools.partial(
    pl.kernel,
    out_type=jax.ShapeDtypeStruct((2, _NP), jnp.float32),
    mesh=_mesh,
    scratch_types=[
        pltpu.VMEM((_NCH, _K), jnp.int32),
        pltpu.VMEM((_EPW,), jnp.float32),
        pltpu.VMEM((_RPT,), jnp.float32),
        pltpu.VMEM_SHARED((_NP,), jnp.float32),
    ],
)
def _sc_degree(dst_hbm, ew_hbm, out_hbm, dst_v, ew_v, zbuf, acc):
    c = lax.axis_index("c")
    s = lax.axis_index("s")
    wid = c * 16 + s
    pltpu.sync_copy(dst_hbm.at[wid], dst_v)
    pltpu.sync_copy(ew_hbm.at[wid], ew_v)

    @pl.loop(0, _RPT // 16)
    def _(i):
        zbuf[pl.ds(i * 16, 16)] = jnp.zeros((16,), jnp.float32)

    pltpu.sync_copy(zbuf, acc.at[pl.ds(s * _RPT, _RPT)])
    plsc.subcore_barrier()

    @pl.loop(0, _NCH)
    def _(j):
        pltpu.sync_copy(ew_v.at[pl.ds(j * _K, _K)],
                        acc.at[dst_v.at[j]], add=True)

    plsc.subcore_barrier()
    pltpu.sync_copy(acc.at[pl.ds(s * _RPT, _RPT)],
                    out_hbm.at[c, pl.ds(s * _RPT, _RPT)])


_NGR = 10
_GC = _NCH // _NGR


def _make_sc_msg(F):
    nf = F // 16

    @functools.partial(
        pl.kernel,
        out_type=jax.ShapeDtypeStruct((2, _NP, F), jnp.float32),
        mesh=_mesh,
        scratch_types=[
            pltpu.VMEM((2, _GC, _K), jnp.int32),
            pltpu.VMEM((2, _GC, _K), jnp.int32),
            pltpu.VMEM((_EPW,), jnp.float32),
            pltpu.VMEM((_K, F), jnp.float32),
            pltpu.VMEM((_K, F), jnp.float32),
            pltpu.VMEM_SHARED((_NP, F), jnp.float32),
            pltpu.SemaphoreType.DMA,
        ],
        compiler_params=pltpu.CompilerParams(use_tc_tiling_on_sc=False),
    )
    def msg(h_hbm, src_hbm, dst_hbm, ew_hbm, out_hbm,
            src_s, dst_s, ew_v, buf0, buf1, acc, sem):
        c = lax.axis_index("c")
        s = lax.axis_index("s")
        wid = c * 16 + s
        bufs = (buf0, buf1)

        def stage(g):
            pltpu.sync_copy(src_hbm.at[wid, g], src_s.at[g % 2])
            pltpu.sync_copy(dst_hbm.at[wid, g], dst_s.at[g % 2])

        def gather(j, b, start):
            d = pltpu.make_async_copy(
                h_hbm.at[src_s.at[(j // _GC) % 2, j % _GC]], bufs[b], sem)
            if start:
                d.start()
            else:
                d.wait()

        pltpu.sync_copy(ew_hbm.at[wid], ew_v)

        @pl.loop(0, _K)
        def _(i):
            for f in range(nf):
                buf0[i, pl.ds(16 * f, 16)] = jnp.zeros((16,), jnp.float32)

        for r in range(_RPT // _K):
            pltpu.sync_copy(buf0, acc.at[pl.ds(s * _RPT + r * _K, _K)])
        plsc.subcore_barrier()

        stage(0)
        gather(0, 0, True)
        gather(1, 1, True)

        @pl.loop(0, _NCH, step=2)
        def _(j0):
            @pl.when(jnp.logical_and(j0 % _GC == _GC - 2,
                                     j0 < _NCH - _GC))
            def _():
                stage(j0 // _GC + 1)

            for b in range(2):
                j = j0 + b
                buf = bufs[b]
                gather(j, b, False)

                for g in range(_K // 16):
                    wvec = ew_v[pl.ds(j * _K + g * 16, 16)]
                    for l in range(16):
                        wsp = jnp.take_along_axis(
                            wvec, jnp.full((16,), l, jnp.int32), axis=0,
                            mode="promise_in_bounds")
                        e = g * 16 + l
                        for f in range(nf):
                            buf[e, pl.ds(16 * f, 16)] = (
                                buf[e, pl.ds(16 * f, 16)] * wsp)

                pltpu.sync_copy(
                    buf, acc.at[dst_s.at[(j // _GC) % 2, j % _GC]], add=True)

                @pl.when(j < _NCH - 2)
                def _():
                    gather(j + 2, b, True)

        plsc.subcore_barrier()
        pltpu.sync_copy(acc.at[pl.ds(s * _RPT, _RPT)],
                        out_hbm.at[c, pl.ds(s * _RPT, _RPT)])

    return msg


_sc_msg128 = _make_sc_msg(_HID)
_sc_msg16 = _make_sc_msg(_C)


def kernel(x, edge_index, epoch, Q, W1, b1, W2, b2,
           conv1_w, conv1_b, conv2_w, conv2_b):
    src = edge_index[0]
    dst = edge_index[1]
    npad = _EP - _E
    pad_idx = jnp.arange(npad, dtype=jnp.int32) % _N
    src_p = jnp.concatenate([src, pad_idx]).reshape(_NW, _NCH, _K)
    dst_p = jnp.concatenate([dst, pad_idx]).reshape(_NW, _NCH, _K)

    w = _edge_mlp(Q, W1, b1, W2, b2)
    ew = jnp.concatenate(
        [w, w, jnp.zeros((npad,), jnp.float32)]).reshape(_NW, _EPW)

    xp = jnp.pad(x, ((0, _NP - _N), (0, 0)))
    xw = _dense_matmul(xp, conv1_w)

    src_g = src_p.reshape(_NW, _NGR, _GC, _K)
    dst_g = dst_p.reshape(_NW, _NGR, _GC, _K)

    degp = _sc_degree(dst_p, ew)
    dinv, h1s = _dinv_and_scale(degp, xw)
    acc1 = _sc_msg128(h1s, src_g, dst_g, ew)
    h2s = _layer2_dense(acc1, dinv, conv1_b, conv2_w)
    acc2 = _sc_msg16(h2s, src_g, dst_g, ew)
    outp = _final_out(acc2, dinv, conv2_b)
    return (outp[:_N], Q)


# speedup vs baseline: 1.2942x; 1.2942x over previous
import functools

import jax
import jax.numpy as jnp
from jax import lax
from jax.experimental import pallas as pl
from jax.experimental.pallas import tpu as pltpu
from jax.experimental.pallas import tpu_sc as plsc

_N = 10000
_NP = 10240
_E = 320000
_EH = 160000
_EP = 327680
_NW = 32
_NCH = 80
_K = 128
_EPW = _NCH * _K
_HID = 128
_C = 16
_RPT = _NP // 16

_mesh = plsc.VectorSubcoreMesh(core_axis_name="c", subcore_axis_name="s")


def _edge_mlp(Q, W1, b1, W2, b2):
    QT = Q.T
    W1T = W1.T
    W2T = W2.T
    G = 5
    R = _EH // G

    def body(q_ref, w1_ref, b1_ref, w2_ref, b2_ref, o_ref):
        h = jnp.dot(w1_ref[...], q_ref[...], preferred_element_type=jnp.float32)
        h = jnp.maximum(h + b1_ref[...][:, None], 0.0)
        o = jnp.dot(w2_ref[...], h, preferred_element_type=jnp.float32)
        o_ref[...] = jax.nn.sigmoid(o + b2_ref[0])

    out = pl.pallas_call(
        body,
        grid=(G,),
        in_specs=[
            pl.BlockSpec((_C, R), lambda i: (0, i)),
            pl.BlockSpec((32, _C), lambda i: (0, 0)),
            pl.BlockSpec((32,), lambda i: (0,)),
            pl.BlockSpec((1, 32), lambda i: (0, 0)),
            pl.BlockSpec((1,), lambda i: (0,)),
        ],
        out_specs=pl.BlockSpec((1, R), lambda i: (0, i)),
        out_shape=jax.ShapeDtypeStruct((1, _EH), jnp.float32),
    )(QT, W1T, b1, W2T, b2)
    return out[0]


def _dense_matmul(xp, w):
    H = w.shape[1]

    def body(x_ref, w_ref, o_ref):
        o_ref[...] = jnp.dot(x_ref[...], w_ref[...],
                             preferred_element_type=jnp.float32)

    return pl.pallas_call(
        body,
        out_shape=jax.ShapeDtypeStruct((_NP, H), jnp.float32),
    )(xp, w)


def _dinv_and_scale(degp, xw):
    def body(d_ref, xw_ref, dinv_ref, h_ref):
        deg = d_ref[0] + d_ref[1]
        dinv = jnp.where(deg > 0,
                         lax.rsqrt(jnp.maximum(deg, 1e-12)),
                         0.0)
        dinv_ref[...] = dinv
        h_ref[...] = xw_ref[...] * dinv[:, None]

    return pl.pallas_call(
        body,
        out_shape=[
            jax.ShapeDtypeStruct((_NP,), jnp.float32),
            jax.ShapeDtypeStruct((_NP, _HID), jnp.float32),
        ],
    )(degp, xw)


def _layer2_dense(acc1, dinv, c1b, w2):
    def body(a_ref, d_ref, b_ref, w_ref, o_ref):
        s = a_ref[0] + a_ref[1]
        dinv = d_ref[...]
        h = jnp.maximum(s * dinv[:, None] + b_ref[...][None, :], 0.0)
        h2 = jnp.dot(h, w_ref[...], preferred_element_type=jnp.float32)
        o_ref[...] = h2 * dinv[:, None]

    return pl.pallas_call(
        body,
        out_shape=jax.ShapeDtypeStruct((_NP, _C), jnp.float32),
    )(acc1, dinv, c1b, w2)


def _final_out(acc2, dinv, c2b):
    def body(a_ref, d_ref, b_ref, o_ref):
        s = a_ref[0] + a_ref[1]
        o_ref[...] = s * d_ref[...][:, None] + b_ref[...][None, :]

    return pl.pallas_call(
        body,
        out_shape=jax.ShapeDtypeStruct((_NP, _C), jnp.float32),
    )(acc2, dinv, c2b)


@functools.partial(
    pl.kernel,
    out_type=jax.ShapeDtypeStruct((2, _NP), jnp.float32),
    mesh=_mesh,
    scratch_types=[
        pltpu.VMEM((_NCH, _K), jnp.int32),
        pltpu.VMEM((_EPW,), jnp.float32),
        pltpu.VMEM((_RPT,), jnp.float32),
        pltpu.VMEM_SHARED((_NP,), jnp.float32),
    ],
)
def _sc_degree(dst_hbm, ew_hbm, out_hbm, dst_v, ew_v, zbuf, acc):
    c = lax.axis_index("c")
    s = lax.axis_index("s")
    wid = c * 16 + s
    pltpu.sync_copy(dst_hbm.at[wid], dst_v)
    pltpu.sync_copy(ew_hbm.at[wid], ew_v)

    @pl.loop(0, _RPT // 16)
    def _(i):
        zbuf[pl.ds(i * 16, 16)] = jnp.zeros((16,), jnp.float32)

    pltpu.sync_copy(zbuf, acc.at[pl.ds(s * _RPT, _RPT)])
    plsc.subcore_barrier()

    @pl.loop(0, _NCH)
    def _(j):
        pltpu.sync_copy(ew_v.at[pl.ds(j * _K, _K)],
                        acc.at[dst_v.at[j]], add=True)

    plsc.subcore_barrier()
    pltpu.sync_copy(acc.at[pl.ds(s * _RPT, _RPT)],
                    out_hbm.at[c, pl.ds(s * _RPT, _RPT)])


_NGR = 10
_GC = _NCH // _NGR


def _make_sc_msg(F):
    nf = F // 16

    @functools.partial(
        pl.kernel,
        out_type=jax.ShapeDtypeStruct((2, _NP, F), jnp.float32),
        mesh=_mesh,
        scratch_types=[
            pltpu.VMEM((2, _GC, _K), jnp.int32),
            pltpu.VMEM((2, _GC, _K), jnp.int32),
            pltpu.VMEM((_EPW,), jnp.float32),
            pltpu.VMEM((_K, F), jnp.float32),
            pltpu.VMEM((_K, F), jnp.float32),
            pltpu.VMEM_SHARED((_NP, F), jnp.float32),
            pltpu.SemaphoreType.DMA,
        ],
        compiler_params=pltpu.CompilerParams(use_tc_tiling_on_sc=False),
    )
    def msg(h_hbm, src_hbm, dst_hbm, ew_hbm, out_hbm,
            src_s, dst_s, ew_v, buf0, buf1, acc, sem):
        c = lax.axis_index("c")
        s = lax.axis_index("s")
        wid = c * 16 + s
        bufs = (buf0, buf1)

        def stage(g):
            pltpu.sync_copy(src_hbm.at[wid, g], src_s.at[g % 2])
            pltpu.sync_copy(dst_hbm.at[wid, g], dst_s.at[g % 2])

        def gather(j, b, start):
            d = pltpu.make_async_copy(
                h_hbm.at[src_s.at[(j // _GC) % 2, j % _GC]], bufs[b], sem)
            if start:
                d.start()
            else:
                d.wait()

        pltpu.sync_copy(ew_hbm.at[wid], ew_v)

        @pl.loop(0, _K)
        def _(i):
            for f in range(nf):
                buf0[i, pl.ds(16 * f, 16)] = jnp.zeros((16,), jnp.float32)

        for r in range(_RPT // _K):
            pltpu.sync_copy(buf0, acc.at[pl.ds(s * _RPT + r * _K, _K)])
        plsc.subcore_barrier()

        stage(0)
        gather(0, 0, True)
        gather(1, 1, True)

        @pl.loop(0, _NCH, step=2)
        def _(j0):
            @pl.when(jnp.logical_and(j0 % _GC == _GC - 2,
                                     j0 < _NCH - _GC))
            def _():
                stage(j0 // _GC + 1)

            for b in range(2):
                j = j0 + b
                buf = bufs[b]
                gather(j, b, False)

                @pl.loop(0, _K // 16, unroll=2)
                def _(g):
                    wvec = ew_v[pl.ds(j * _K + g * 16, 16)]
                    for l in range(16):
                        wsp = jnp.take_along_axis(
                            wvec, jnp.full((16,), l, jnp.int32), axis=0,
                            mode="promise_in_bounds")
                        e = g * 16 + l
                        for f in range(nf):
                            buf[e, pl.ds(16 * f, 16)] = (
                                buf[e, pl.ds(16 * f, 16)] * wsp)

                pltpu.sync_copy(
                    buf, acc.at[dst_s.at[(j // _GC) % 2, j % _GC]], add=True)

                @pl.when(j < _NCH - 2)
                def _():
                    gather(j + 2, b, True)

        plsc.subcore_barrier()
        pltpu.sync_copy(acc.at[pl.ds(s * _RPT, _RPT)],
                        out_hbm.at[c, pl.ds(s * _RPT, _RPT)])

    return msg


_sc_msg128 = _make_sc_msg(_HID)
_sc_msg16 = _make_sc_msg(_C)


def kernel(x, edge_index, epoch, Q, W1, b1, W2, b2,
           conv1_w, conv1_b, conv2_w, conv2_b):
    src = edge_index[0]
    dst = edge_index[1]
    npad = _EP - _E
    pad_idx = jnp.arange(npad, dtype=jnp.int32) % _N
    src_p = jnp.concatenate([src, pad_idx]).reshape(_NW, _NCH, _K)
    dst_p = jnp.concatenate([dst, pad_idx]).reshape(_NW, _NCH, _K)

    w = _edge_mlp(Q, W1, b1, W2, b2)
    ew = jnp.concatenate(
        [w, w, jnp.zeros((npad,), jnp.float32)]).reshape(_NW, _EPW)

    xp = jnp.pad(x, ((0, _NP - _N), (0, 0)))
    xw = _dense_matmul(xp, conv1_w)

    src_g = src_p.reshape(_NW, _NGR, _GC, _K)
    dst_g = dst_p.reshape(_NW, _NGR, _GC, _K)

    degp = _sc_degree(dst_p, ew)
    dinv, h1s = _dinv_and_scale(degp, xw)
    acc1 = _sc_msg128(h1s, src_g, dst_g, ew)
    h2s = _layer2_dense(acc1, dinv, conv1_b, conv2_w)
    acc2 = _sc_msg16(h2s, src_g, dst_g, ew)
    outp = _final_out(acc2, dinv, conv2_b)
    return (outp[:_N], Q)


# speedup vs baseline: 1.3091x; 1.0115x over previous
import functools

import jax
import jax.numpy as jnp
from jax import lax
from jax.experimental import pallas as pl
from jax.experimental.pallas import tpu as pltpu
from jax.experimental.pallas import tpu_sc as plsc

_N = 10000
_NP = 10240
_E = 320000
_EH = 160000
_EP = 327680
_NW = 32
_NCH = 80
_K = 128
_EPW = _NCH * _K
_HID = 128
_C = 16
_RPT = _NP // 16

_mesh = plsc.VectorSubcoreMesh(core_axis_name="c", subcore_axis_name="s")


def _edge_mlp(Q, W1, b1, W2, b2):
    QT = Q.T
    W1T = W1.T
    W2T = W2.T
    G = 5
    R = _EH // G

    def body(q_ref, w1_ref, b1_ref, w2_ref, b2_ref, o_ref):
        h = jnp.dot(w1_ref[...], q_ref[...], preferred_element_type=jnp.float32)
        h = jnp.maximum(h + b1_ref[...][:, None], 0.0)
        o = jnp.dot(w2_ref[...], h, preferred_element_type=jnp.float32)
        o_ref[...] = jax.nn.sigmoid(o + b2_ref[0])

    out = pl.pallas_call(
        body,
        grid=(G,),
        in_specs=[
            pl.BlockSpec((_C, R), lambda i: (0, i)),
            pl.BlockSpec((32, _C), lambda i: (0, 0)),
            pl.BlockSpec((32,), lambda i: (0,)),
            pl.BlockSpec((1, 32), lambda i: (0, 0)),
            pl.BlockSpec((1,), lambda i: (0,)),
        ],
        out_specs=pl.BlockSpec((1, R), lambda i: (0, i)),
        out_shape=jax.ShapeDtypeStruct((1, _EH), jnp.float32),
    )(QT, W1T, b1, W2T, b2)
    return out[0]


def _dense_matmul(xp, w):
    H = w.shape[1]

    def body(x_ref, w_ref, o_ref):
        o_ref[...] = jnp.dot(x_ref[...], w_ref[...],
                             preferred_element_type=jnp.float32)

    return pl.pallas_call(
        body,
        out_shape=jax.ShapeDtypeStruct((_NP, H), jnp.float32),
    )(xp, w)


def _dinv_and_scale(degp, xw):
    def body(d_ref, xw_ref, dinv_ref, h_ref):
        deg = d_ref[0] + d_ref[1]
        dinv = jnp.where(deg > 0,
                         lax.rsqrt(jnp.maximum(deg, 1e-12)),
                         0.0)
        dinv_ref[...] = dinv
        h_ref[...] = xw_ref[...] * dinv[:, None]

    return pl.pallas_call(
        body,
        out_shape=[
            jax.ShapeDtypeStruct((_NP,), jnp.float32),
            jax.ShapeDtypeStruct((_NP, _HID), jnp.float32),
        ],
    )(degp, xw)


def _layer2_dense(acc1, dinv, c1b, w2):
    def body(a_ref, d_ref, b_ref, w_ref, o_ref):
        s = a_ref[0] + a_ref[1]
        dinv = d_ref[...]
        h = jnp.maximum(s * dinv[:, None] + b_ref[...][None, :], 0.0)
        h2 = jnp.dot(h, w_ref[...], preferred_element_type=jnp.float32)
        o_ref[...] = h2 * dinv[:, None]

    return pl.pallas_call(
        body,
        out_shape=jax.ShapeDtypeStruct((_NP, _C), jnp.float32),
    )(acc1, dinv, c1b, w2)


def _final_out(acc2, dinv, c2b):
    def body(a_ref, d_ref, b_ref, o_ref):
        s = a_ref[0] + a_ref[1]
        o_ref[...] = s * d_ref[...][:, None] + b_ref[...][None, :]

    return pl.pallas_call(
        body,
        out_shape=jax.ShapeDtypeStruct((_NP, _C), jnp.float32),
    )(acc2, dinv, c2b)


@functools.partial(
    pl.kernel,
    out_type=jax.ShapeDtypeStruct((2, _NP), jnp.float32),
    mesh=_mesh,
    scratch_types=[
        pltpu.VMEM((_NCH, _K), jnp.int32),
        pltpu.VMEM((_EPW,), jnp.float32),
        pltpu.VMEM((_RPT,), jnp.float32),
        pltpu.VMEM_SHARED((_NP,), jnp.float32),
    ],
)
def _sc_degree(dst_hbm, ew_hbm, out_hbm, dst_v, ew_v, zbuf, acc):
    c = lax.axis_index("c")
    s = lax.axis_index("s")
    wid = c * 16 + s
    pltpu.sync_copy(dst_hbm.at[wid], dst_v)
    pltpu.sync_copy(ew_hbm.at[wid], ew_v)

    @pl.loop(0, _RPT // 16)
    def _(i):
        zbuf[pl.ds(i * 16, 16)] = jnp.zeros((16,), jnp.float32)

    pltpu.sync_copy(zbuf, acc.at[pl.ds(s * _RPT, _RPT)])
    plsc.subcore_barrier()

    @pl.loop(0, _NCH)
    def _(j):
        pltpu.sync_copy(ew_v.at[pl.ds(j * _K, _K)],
                        acc.at[dst_v.at[j]], add=True)

    plsc.subcore_barrier()
    pltpu.sync_copy(acc.at[pl.ds(s * _RPT, _RPT)],
                    out_hbm.at[c, pl.ds(s * _RPT, _RPT)])


_KM = 64
_NCHM = _EPW // _KM
_GCM = 8
_NGRM = _NCHM // _GCM
_NB = 4


def _make_sc_msg(F):
    nf = F // 16

    @functools.partial(
        pl.kernel,
        out_type=jax.ShapeDtypeStruct((2, _NP, F), jnp.float32),
        mesh=_mesh,
        scratch_types=[
            pltpu.VMEM((2, _GCM, _KM), jnp.int32),
            pltpu.VMEM((2, _GCM, _KM), jnp.int32),
            pltpu.VMEM((_EPW,), jnp.float32),
            pltpu.VMEM((_KM, F), jnp.float32),
            pltpu.VMEM((_KM, F), jnp.float32),
            pltpu.VMEM((_KM, F), jnp.float32),
            pltpu.VMEM((_KM, F), jnp.float32),
            pltpu.VMEM_SHARED((_NP, F), jnp.float32),
            pltpu.SemaphoreType.DMA,
            pltpu.SemaphoreType.DMA,
        ],
        compiler_params=pltpu.CompilerParams(use_tc_tiling_on_sc=False),
    )
    def msg(h_hbm, src_hbm, dst_hbm, ew_hbm, out_hbm,
            src_s, dst_s, ew_v, buf0, buf1, buf2, buf3, acc, sem, ssem):
        c = lax.axis_index("c")
        s = lax.axis_index("s")
        wid = c * 16 + s
        bufs = (buf0, buf1, buf2, buf3)

        def stage(g):
            pltpu.sync_copy(src_hbm.at[wid, g], src_s.at[g % 2])
            pltpu.sync_copy(dst_hbm.at[wid, g], dst_s.at[g % 2])

        def gather(j, b, start):
            d = pltpu.make_async_copy(
                h_hbm.at[src_s.at[(j // _GCM) % 2, j % _GCM]], bufs[b], sem)
            if start:
                d.start()
            else:
                d.wait()

        def drain_scatter(b):
            pltpu.make_async_copy(h_hbm.at[pl.ds(0, _KM)], bufs[b], ssem).wait()

        pltpu.sync_copy(ew_hbm.at[wid], ew_v)

        @pl.loop(0, _KM)
        def _(i):
            for f in range(nf):
                buf0[i, pl.ds(16 * f, 16)] = jnp.zeros((16,), jnp.float32)

        for r in range(_RPT // _KM):
            pltpu.sync_copy(buf0, acc.at[pl.ds(s * _RPT + r * _KM, _KM)])
        plsc.subcore_barrier()

        stage(0)
        gather(0, 0, True)
        gather(1, 1, True)

        @pl.loop(0, _NCHM, step=_NB)
        def _(j0):
            for b in range(_NB):
                j = j0 + b

                @pl.when(jnp.logical_and(j % _GCM == _GCM - 2,
                                         j < _NCHM - 2))
                def _():
                    stage(j // _GCM + 1)

                @pl.when(j >= 2)
                def _():
                    drain_scatter((b + 2) % _NB)

                @pl.when(j < _NCHM - 2)
                def _():
                    gather(j + 2, (b + 2) % _NB, True)

                buf = bufs[b]
                gather(j, b, False)

                @pl.loop(0, _KM // 16, unroll=2)
                def _(g):
                    wvec = ew_v[pl.ds(j * _KM + g * 16, 16)]
                    for l in range(16):
                        wsp = jnp.take_along_axis(
                            wvec, jnp.full((16,), l, jnp.int32), axis=0,
                            mode="promise_in_bounds")
                        e = g * 16 + l
                        for f in range(nf):
                            buf[e, pl.ds(16 * f, 16)] = (
                                buf[e, pl.ds(16 * f, 16)] * wsp)

                pltpu.async_copy(
                    buf, acc.at[dst_s.at[(j // _GCM) % 2, j % _GCM]], ssem,
                    add=True)

        drain_scatter((_NCHM - 2) % _NB)
        drain_scatter((_NCHM - 1) % _NB)
        plsc.subcore_barrier()
        pltpu.sync_copy(acc.at[pl.ds(s * _RPT, _RPT)],
                        out_hbm.at[c, pl.ds(s * _RPT, _RPT)])

    return msg


_sc_msg128 = _make_sc_msg(_HID)
_sc_msg16 = _make_sc_msg(_C)


def kernel(x, edge_index, epoch, Q, W1, b1, W2, b2,
           conv1_w, conv1_b, conv2_w, conv2_b):
    src = edge_index[0]
    dst = edge_index[1]
    npad = _EP - _E
    pad_idx = jnp.arange(npad, dtype=jnp.int32) % _N
    src_p = jnp.concatenate([src, pad_idx]).reshape(_NW, _NCH, _K)
    dst_p = jnp.concatenate([dst, pad_idx]).reshape(_NW, _NCH, _K)

    w = _edge_mlp(Q, W1, b1, W2, b2)
    ew = jnp.concatenate(
        [w, w, jnp.zeros((npad,), jnp.float32)]).reshape(_NW, _EPW)

    xp = jnp.pad(x, ((0, _NP - _N), (0, 0)))
    xw = _dense_matmul(xp, conv1_w)

    src_g = src_p.reshape(_NW, _NGRM, _GCM, _KM)
    dst_g = dst_p.reshape(_NW, _NGRM, _GCM, _KM)

    degp = _sc_degree(dst_p, ew)
    dinv, h1s = _dinv_and_scale(degp, xw)
    acc1 = _sc_msg128(h1s, src_g, dst_g, ew)
    h2s = _layer2_dense(acc1, dinv, conv1_b, conv2_w)
    acc2 = _sc_msg16(h2s, src_g, dst_g, ew)
    outp = _final_out(acc2, dinv, conv2_b)
    return (outp[:_N], Q)


# speedup vs baseline: 1.4101x; 1.0771x over previous
import functools

import jax
import jax.numpy as jnp
from jax import lax
from jax.experimental import pallas as pl
from jax.experimental.pallas import tpu as pltpu
from jax.experimental.pallas import tpu_sc as plsc

_N = 10000
_NP = 10240
_E = 320000
_EH = 160000
_EP = 327680
_NW = 32
_NCH = 80
_K = 128
_EPW = _NCH * _K
_HID = 128
_C = 16
_RPT = _NP // 16

_mesh = plsc.VectorSubcoreMesh(core_axis_name="c", subcore_axis_name="s")


def _edge_mlp(Q, W1, b1, W2, b2):
    QT = Q.T
    W1T = W1.T
    W2T = W2.T
    G = 5
    R = _EH // G

    def body(q_ref, w1_ref, b1_ref, w2_ref, b2_ref, o_ref):
        h = jnp.dot(w1_ref[...], q_ref[...], preferred_element_type=jnp.float32)
        h = jnp.maximum(h + b1_ref[...][:, None], 0.0)
        o = jnp.dot(w2_ref[...], h, preferred_element_type=jnp.float32)
        o_ref[...] = jax.nn.sigmoid(o + b2_ref[0])

    out = pl.pallas_call(
        body,
        grid=(G,),
        in_specs=[
            pl.BlockSpec((_C, R), lambda i: (0, i)),
            pl.BlockSpec((32, _C), lambda i: (0, 0)),
            pl.BlockSpec((32,), lambda i: (0,)),
            pl.BlockSpec((1, 32), lambda i: (0, 0)),
            pl.BlockSpec((1,), lambda i: (0,)),
        ],
        out_specs=pl.BlockSpec((1, R), lambda i: (0, i)),
        out_shape=jax.ShapeDtypeStruct((1, _EH), jnp.float32),
    )(QT, W1T, b1, W2T, b2)
    return out[0]


def _dense_matmul(xp, w):
    H = w.shape[1]

    def body(x_ref, w_ref, o_ref):
        o_ref[...] = jnp.dot(x_ref[...], w_ref[...],
                             preferred_element_type=jnp.float32)

    return pl.pallas_call(
        body,
        out_shape=jax.ShapeDtypeStruct((_NP, H), jnp.float32),
    )(xp, w)


def _dinv_and_scale(degp, xw):
    def body(d_ref, xw_ref, dinv_ref, h_ref):
        deg = d_ref[0] + d_ref[1]
        dinv = jnp.where(deg > 0,
                         lax.rsqrt(jnp.maximum(deg, 1e-12)),
                         0.0)
        dinv_ref[...] = dinv
        h_ref[...] = xw_ref[...] * dinv[:, None]

    return pl.pallas_call(
        body,
        out_shape=[
            jax.ShapeDtypeStruct((_NP,), jnp.float32),
            jax.ShapeDtypeStruct((_NP, _HID), jnp.float32),
        ],
    )(degp, xw)


def _layer2_dense(acc1, dinv, c1b, w2):
    def body(a_ref, d_ref, b_ref, w_ref, o_ref):
        s = a_ref[0] + a_ref[1]
        dinv = d_ref[...]
        h = jnp.maximum(s * dinv[:, None] + b_ref[...][None, :], 0.0)
        h2 = jnp.dot(h, w_ref[...], preferred_element_type=jnp.float32)
        o_ref[...] = h2 * dinv[:, None]

    return pl.pallas_call(
        body,
        out_shape=jax.ShapeDtypeStruct((_NP, _C), jnp.float32),
    )(acc1, dinv, c1b, w2)


def _final_out(acc2, dinv, c2b):
    def body(a_ref, d_ref, b_ref, o_ref):
        s = a_ref[0] + a_ref[1]
        o_ref[...] = s * d_ref[...][:, None] + b_ref[...][None, :]

    return pl.pallas_call(
        body,
        out_shape=jax.ShapeDtypeStruct((_NP, _C), jnp.float32),
    )(acc2, dinv, c2b)


@functools.partial(
    pl.kernel,
    out_type=jax.ShapeDtypeStruct((2, _NP), jnp.float32),
    mesh=_mesh,
    scratch_types=[
        pltpu.VMEM((_NCH, _K), jnp.int32),
        pltpu.VMEM((_EPW,), jnp.float32),
        pltpu.VMEM((_RPT,), jnp.float32),
        pltpu.VMEM_SHARED((_NP,), jnp.float32),
    ],
)
def _sc_degree(dst_hbm, ew_hbm, out_hbm, dst_v, ew_v, zbuf, acc):
    c = lax.axis_index("c")
    s = lax.axis_index("s")
    wid = c * 16 + s
    pltpu.sync_copy(dst_hbm.at[wid], dst_v)
    pltpu.sync_copy(ew_hbm.at[wid], ew_v)

    @pl.loop(0, _RPT // 16)
    def _(i):
        zbuf[pl.ds(i * 16, 16)] = jnp.zeros((16,), jnp.float32)

    pltpu.sync_copy(zbuf, acc.at[pl.ds(s * _RPT, _RPT)])
    plsc.subcore_barrier()

    @pl.loop(0, _NCH)
    def _(j):
        pltpu.sync_copy(ew_v.at[pl.ds(j * _K, _K)],
                        acc.at[dst_v.at[j]], add=True)

    plsc.subcore_barrier()
    pltpu.sync_copy(acc.at[pl.ds(s * _RPT, _RPT)],
                    out_hbm.at[c, pl.ds(s * _RPT, _RPT)])


_GCM = 8
_NB = 4


def _make_sc_msg(F, _KM, tc_tiling):
    _NCHM = _EPW // _KM
    _NGRM = _NCHM // _GCM
    nf = F // 16

    @functools.partial(
        pl.kernel,
        out_type=jax.ShapeDtypeStruct((2, _NP, F), jnp.float32),
        mesh=_mesh,
        scratch_types=[
            pltpu.VMEM((2, _GCM, _KM), jnp.int32),
            pltpu.VMEM((2, _GCM, _KM), jnp.int32),
            pltpu.VMEM((_EPW,), jnp.float32),
            pltpu.VMEM((_KM, F), jnp.float32),
            pltpu.VMEM((_KM, F), jnp.float32),
            pltpu.VMEM((_KM, F), jnp.float32),
            pltpu.VMEM((_KM, F), jnp.float32),
            pltpu.VMEM_SHARED((_NP, F), jnp.float32),
            pltpu.SemaphoreType.DMA,
            pltpu.SemaphoreType.DMA,
        ],
        compiler_params=pltpu.CompilerParams(use_tc_tiling_on_sc=tc_tiling),
    )
    def msg(h_hbm, src_hbm, dst_hbm, ew_hbm, out_hbm,
            src_s, dst_s, ew_v, buf0, buf1, buf2, buf3, acc, sem, ssem):
        c = lax.axis_index("c")
        s = lax.axis_index("s")
        wid = c * 16 + s
        bufs = (buf0, buf1, buf2, buf3)

        def stage(g):
            pltpu.sync_copy(src_hbm.at[wid, g], src_s.at[g % 2])
            pltpu.sync_copy(dst_hbm.at[wid, g], dst_s.at[g % 2])

        def gather(j, b, start):
            d = pltpu.make_async_copy(
                h_hbm.at[src_s.at[(j // _GCM) % 2, j % _GCM]], bufs[b], sem)
            if start:
                d.start()
            else:
                d.wait()

        def drain_scatter(b):
            pltpu.make_async_copy(h_hbm.at[pl.ds(0, _KM)], bufs[b], ssem).wait()

        pltpu.sync_copy(ew_hbm.at[wid], ew_v)

        @pl.loop(0, _KM)
        def _(i):
            for f in range(nf):
                buf0[i, pl.ds(16 * f, 16)] = jnp.zeros((16,), jnp.float32)

        for r in range(_RPT // _KM):
            pltpu.sync_copy(buf0, acc.at[pl.ds(s * _RPT + r * _KM, _KM)])
        plsc.subcore_barrier()

        stage(0)
        gather(0, 0, True)
        gather(1, 1, True)

        @pl.loop(0, _NCHM, step=_NB)
        def _(j0):
            for b in range(_NB):
                j = j0 + b

                @pl.when(jnp.logical_and(j % _GCM == _GCM - 2,
                                         j < _NCHM - 2))
                def _():
                    stage(j // _GCM + 1)

                @pl.when(j >= 2)
                def _():
                    drain_scatter((b + 2) % _NB)

                @pl.when(j < _NCHM - 2)
                def _():
                    gather(j + 2, (b + 2) % _NB, True)

                buf = bufs[b]
                gather(j, b, False)

                @pl.loop(0, _KM // 16, unroll=2)
                def _(g):
                    wvec = ew_v[pl.ds(j * _KM + g * 16, 16)]
                    for l in range(16):
                        wsp = jnp.take_along_axis(
                            wvec, jnp.full((16,), l, jnp.int32), axis=0,
                            mode="promise_in_bounds")
                        e = g * 16 + l
                        for f in range(nf):
                            buf[e, pl.ds(16 * f, 16)] = (
                                buf[e, pl.ds(16 * f, 16)] * wsp)

                pltpu.async_copy(
                    buf, acc.at[dst_s.at[(j // _GCM) % 2, j % _GCM]], ssem,
                    add=True)

        drain_scatter((_NCHM - 2) % _NB)
        drain_scatter((_NCHM - 1) % _NB)
        plsc.subcore_barrier()
        pltpu.sync_copy(acc.at[pl.ds(s * _RPT, _RPT)],
                        out_hbm.at[c, pl.ds(s * _RPT, _RPT)])

    return msg


_sc_msg128 = _make_sc_msg(_HID, 64, True)
_sc_msg16 = _make_sc_msg(_C, 128, False)


def kernel(x, edge_index, epoch, Q, W1, b1, W2, b2,
           conv1_w, conv1_b, conv2_w, conv2_b):
    src = edge_index[0]
    dst = edge_index[1]
    npad = _EP - _E
    pad_idx = jnp.arange(npad, dtype=jnp.int32) % _N
    src_p = jnp.concatenate([src, pad_idx]).reshape(_NW, _NCH, _K)
    dst_p = jnp.concatenate([dst, pad_idx]).reshape(_NW, _NCH, _K)

    w = _edge_mlp(Q, W1, b1, W2, b2)
    ew = jnp.concatenate(
        [w, w, jnp.zeros((npad,), jnp.float32)]).reshape(_NW, _EPW)

    xp = jnp.pad(x, ((0, _NP - _N), (0, 0)))
    xw = _dense_matmul(xp, conv1_w)

    src_g = src_p.reshape(_NW, _EPW // (_GCM * 64), _GCM, 64)
    dst_g = dst_p.reshape(_NW, _EPW // (_GCM * 64), _GCM, 64)
    src_g2 = src_p.reshape(_NW, _EPW // (_GCM * 128), _GCM, 128)
    dst_g2 = dst_p.reshape(_NW, _EPW // (_GCM * 128), _GCM, 128)

    degp = _sc_degree(dst_p, ew)
    dinv, h1s = _dinv_and_scale(degp, xw)
    acc1 = _sc_msg128(h1s, src_g, dst_g, ew)
    h2s = _layer2_dense(acc1, dinv, conv1_b, conv2_w)
    acc2 = _sc_msg16(h2s, src_g2, dst_g2, ew)
    outp = _final_out(acc2, dinv, conv2_b)
    return (outp[:_N], Q)
